# baseline (speedup 1.0000x reference)
import functools

import jax
import jax.numpy as jnp
from jax import lax
from jax.experimental import pallas as pl
from jax.experimental.pallas import tpu as pltpu
from jax.experimental.pallas import tpu_sc as plsc

LANES = 16
NBUF = 4


@functools.lru_cache(maxsize=None)
def _make_gather_sum(seq, batch, vocab, dim):
    info = plsc.get_sparse_core_info()
    nc, ns = info.num_cores, info.num_subcores
    nw = nc * ns
    assert batch % nw == 0
    bpw = batch // nw
    vpr = dim // LANES
    assert seq % NBUF == 0

    mesh = plsc.VectorSubcoreMesh(core_axis_name="c", subcore_axis_name="s")

    @functools.partial(
        pl.kernel,
        mesh=mesh,
        out_type=jax.ShapeDtypeStruct((batch, dim), jnp.float32),
        scratch_types=[
            pltpu.VMEM((seq, bpw), jnp.int32),
            pltpu.VMEM((NBUF, bpw, dim), jnp.float32),
            pltpu.VMEM((bpw, dim), jnp.float32),
        ] + [pltpu.SemaphoreType.DMA] * NBUF,
    )
    def gather_sum(idx_hbm, table_hbm, out_hbm, idx_v, rows_v, acc_v,
                   *sems):
        wid = lax.axis_index("s") * nc + lax.axis_index("c")
        base = wid * bpw

        pltpu.sync_copy(idx_hbm.at[:, pl.ds(base, bpw)], idx_v)

        def zrow(j, carry):
            for v in range(vpr):
                acc_v[j, pl.ds(v * LANES, LANES)] = jnp.zeros(
                    (LANES,), jnp.float32)
            return carry

        lax.fori_loop(0, bpw, zrow, 0, unroll=8)

        def issue(s, b):
            half = bpw // 2
            pltpu.async_copy(table_hbm.at[idx_v.at[s, pl.ds(0, half)]],
                             rows_v.at[b, pl.ds(0, half)], sems[b])
            pltpu.async_copy(table_hbm.at[idx_v.at[s, pl.ds(half, half)]],
                             rows_v.at[b, pl.ds(half, half)], sems[b])

        def wait(b):
            pltpu.make_async_copy(table_hbm.at[pl.ds(0, bpw)],
                                  rows_v.at[b], sems[b]).wait()

        def accum(b):
            def arow(j, carry):
                for v in range(vpr):
                    sl = pl.ds(v * LANES, LANES)
                    plsc.addupdate(acc_v.at[j, sl], rows_v[b, j, sl])
                return carry

            lax.fori_loop(0, bpw, arow, 0, unroll=4)

        for b in range(NBUF - 1):
            issue(b, b)

        def sbody(i, carry):
            s0 = NBUF * i
            for b in range(NBUF):
                s = s0 + b
                nxt = s + NBUF - 1

                @pl.when(nxt < seq)
                def _():
                    issue(nxt, (b + NBUF - 1) % NBUF)

                wait(b)
                accum(b)
            return carry

        lax.fori_loop(0, seq // NBUF, sbody, 0)

        pltpu.sync_copy(acc_v, out_hbm.at[pl.ds(base, bpw)])

    return gather_sum


def _fc_body(x_ref, w_ref, b_ref, o_ref):
    o_ref[...] = lax.dot_general(
        x_ref[...], w_ref[...], (((1,), (1,)), ((), ())),
        preferred_element_type=jnp.float32) + b_ref[...]


@functools.lru_cache(maxsize=None)
def _make_fc(batch, dim, out_dim):
    blk = min(batch, 512)
    return pl.pallas_call(
        _fc_body,
        grid=(batch // blk,),
        in_specs=[
            pl.BlockSpec((blk, dim), lambda i: (i, 0)),
            pl.BlockSpec((out_dim, dim), lambda i: (0, 0)),
            pl.BlockSpec((1, out_dim), lambda i: (0, 0)),
        ],
        out_specs=pl.BlockSpec((blk, out_dim), lambda i: (i, 0)),
        out_shape=jax.ShapeDtypeStruct((batch, out_dim), jnp.float32),
    )


def kernel(text, embedding_table, fc_weight, fc_bias):
    seq, batch = text.shape
    vocab, dim = embedding_table.shape
    out_dim = fc_weight.shape[0]

    idx = text.astype(jnp.int32)
    summed = _make_gather_sum(seq, batch, vocab, dim)(idx, embedding_table)
    fc = _make_fc(batch, dim, out_dim)
    return fc(summed, fc_weight, fc_bias.reshape(1, out_dim))


# speedup vs baseline: 1.2637x; 1.2637x over previous
import functools

import jax
import jax.numpy as jnp
from jax import lax
from jax.experimental import pallas as pl
from jax.experimental.pallas import tpu as pltpu
from jax.experimental.pallas import tpu_sc as plsc

LANES = 16


@functools.lru_cache(maxsize=None)
def _make_gather_sum(seq, batch, vocab, dim):
    info = plsc.get_sparse_core_info()
    nc, ns = info.num_cores, info.num_subcores
    nw = nc * ns
    assert batch % nw == 0
    bpw = batch // nw
    vpr = dim // LANES
    assert bpw % 2 == 0
    assert seq % 8 == 0
    pieces = []
    off = 0
    while off < seq:
        n = min(128, seq - off)
        pieces.append((off, n))
        off += n

    mesh = plsc.VectorSubcoreMesh(core_axis_name="c", subcore_axis_name="s")

    @functools.partial(
        pl.kernel,
        mesh=mesh,
        out_type=jax.ShapeDtypeStruct((batch, dim), jnp.float32),
        scratch_types=[
            pltpu.VMEM((bpw, seq), jnp.int32),
            pltpu.VMEM((2, seq, dim), jnp.float32),
            pltpu.VMEM((bpw, dim), jnp.float32),
            pltpu.SemaphoreType.DMA,
            pltpu.SemaphoreType.DMA,
        ],
    )
    def gather_sum(idxt_hbm, table_hbm, out_hbm, idx_v, rows_v, out_v,
                   sem0, sem1):
        wid = lax.axis_index("s") * nc + lax.axis_index("c")
        base = wid * bpw

        pltpu.sync_copy(idxt_hbm.at[pl.ds(base, bpw)], idx_v)

        sems = (sem0, sem1)

        def issue(lb, buf):
            for off, n in pieces:
                pltpu.async_copy(
                    table_hbm.at[idx_v.at[lb, pl.ds(off, n)]],
                    rows_v.at[buf, pl.ds(off, n)], sems[buf])

        def wait(buf):
            pltpu.make_async_copy(table_hbm.at[pl.ds(0, seq)],
                                  rows_v.at[buf], sems[buf]).wait()

        def accum(lb, buf):
            zero = jnp.zeros((LANES,), jnp.float32)

            def body(r, carry):
                return tuple(
                    carry[v] + rows_v[buf, r, pl.ds(v * LANES, LANES)]
                    for v in range(vpr))

            acc = lax.fori_loop(0, seq, body, (zero,) * vpr, unroll=2)
            for v in range(vpr):
                out_v[lb, pl.ds(v * LANES, LANES)] = acc[v]

        issue(0, 0)

        def bbody(i, carry):
            lb0 = 2 * i
            issue(lb0 + 1, 1)
            wait(0)
            accum(lb0, 0)

            @pl.when(lb0 + 2 < bpw)
            def _():
                issue(lb0 + 2, 0)

            wait(1)
            accum(lb0 + 1, 1)
            return carry

        lax.fori_loop(0, bpw // 2, bbody, 0)

        pltpu.sync_copy(out_v, out_hbm.at[pl.ds(base, bpw)])

    return gather_sum


def _fc_body(x_ref, w_ref, b_ref, o_ref):
    o_ref[...] = lax.dot_general(
        x_ref[...], w_ref[...], (((1,), (1,)), ((), ())),
        preferred_element_type=jnp.float32) + b_ref[...]


@functools.lru_cache(maxsize=None)
def _make_fc(batch, dim, out_dim):
    blk = min(batch, 512)
    return pl.pallas_call(
        _fc_body,
        grid=(batch // blk,),
        in_specs=[
            pl.BlockSpec((blk, dim), lambda i: (i, 0)),
            pl.BlockSpec((out_dim, dim), lambda i: (0, 0)),
            pl.BlockSpec((1, out_dim), lambda i: (0, 0)),
        ],
        out_specs=pl.BlockSpec((blk, out_dim), lambda i: (i, 0)),
        out_shape=jax.ShapeDtypeStruct((batch, out_dim), jnp.float32),
    )


def kernel(text, embedding_table, fc_weight, fc_bias):
    seq, batch = text.shape
    vocab, dim = embedding_table.shape
    out_dim = fc_weight.shape[0]

    idxt = text.astype(jnp.int32).T
    summed = _make_gather_sum(seq, batch, vocab, dim)(idxt, embedding_table)
    fc = _make_fc(batch, dim, out_dim)
    return fc(summed, fc_weight, fc_bias.reshape(1, out_dim))


# speedup vs baseline: 1.5575x; 1.2325x over previous
import functools

import jax
import jax.numpy as jnp
from jax import lax
from jax.experimental import pallas as pl
from jax.experimental.pallas import tpu as pltpu
from jax.experimental.pallas import tpu_sc as plsc

LANES = 16


@functools.lru_cache(maxsize=None)
def _make_gather_sum(seq, batch, vocab, dim):
    info = plsc.get_sparse_core_info()
    nc, ns = info.num_cores, info.num_subcores
    nw = nc * ns
    assert batch % nw == 0
    bpw = batch // nw
    vpr = dim // LANES
    assert bpw % 2 == 0
    assert seq % 8 == 0
    pieces = []
    off = 0
    while off < seq:
        n = min(128, seq - off)
        pieces.append((off, n))
        off += n

    mesh = plsc.VectorSubcoreMesh(core_axis_name="c", subcore_axis_name="s")

    @functools.partial(
        pl.kernel,
        mesh=mesh,
        out_type=jax.ShapeDtypeStruct((batch, dim), jnp.float32),
        scratch_types=[
            pltpu.VMEM((bpw, seq), jnp.int32),
            pltpu.VMEM((3, seq, dim), jnp.float32),
            pltpu.VMEM((bpw, dim), jnp.float32),
            pltpu.SemaphoreType.DMA,
            pltpu.SemaphoreType.DMA,
            pltpu.SemaphoreType.DMA,
        ],
    )
    def gather_sum(idxt_hbm, table_hbm, out_hbm, idx_v, rows_v, out_v,
                   sem0, sem1, sem2):
        wid = lax.axis_index("s") * nc + lax.axis_index("c")
        base = wid * bpw

        pltpu.sync_copy(idxt_hbm.at[pl.ds(base, bpw)], idx_v)

        sems = (sem0, sem1, sem2)

        def issue(lb, buf):
            for off, n in pieces:
                pltpu.async_copy(
                    table_hbm.at[idx_v.at[lb, pl.ds(off, n)]],
                    rows_v.at[buf, pl.ds(off, n)], sems[buf])

        def wait(buf):
            pltpu.make_async_copy(table_hbm.at[pl.ds(0, seq)],
                                  rows_v.at[buf], sems[buf]).wait()

        def accum(lb, buf):
            zero = jnp.zeros((LANES,), jnp.float32)

            def body(r, carry):
                return tuple(
                    carry[v] + rows_v[buf, r, pl.ds(v * LANES, LANES)]
                    for v in range(vpr))

            acc = lax.fori_loop(0, seq, body, (zero,) * vpr, unroll=4)
            for v in range(vpr):
                out_v[lb, pl.ds(v * LANES, LANES)] = acc[v]

        assert bpw % 3 == 2
        issue(0, 0)
        issue(1, 1)

        def bbody(i, carry):
            lb0 = 3 * i
            for b in range(3):
                issue(lb0 + b + 2, (b + 2) % 3)
                wait(b)
                accum(lb0 + b, b)
            return carry

        lax.fori_loop(0, bpw // 3, bbody, 0)
        wait(0)
        accum(bpw - 2, 0)
        wait(1)
        accum(bpw - 1, 1)

        pltpu.sync_copy(out_v, out_hbm.at[pl.ds(base, bpw)])

    return gather_sum


def _fc_body(x_ref, w_ref, b_ref, o_ref):
    o_ref[...] = lax.dot_general(
        x_ref[...], w_ref[...], (((1,), (1,)), ((), ())),
        preferred_element_type=jnp.float32) + b_ref[...]


@functools.lru_cache(maxsize=None)
def _make_fc(batch, dim, out_dim):
    blk = min(batch, 512)
    return pl.pallas_call(
        _fc_body,
        grid=(batch // blk,),
        in_specs=[
            pl.BlockSpec((blk, dim), lambda i: (i, 0)),
            pl.BlockSpec((out_dim, dim), lambda i: (0, 0)),
            pl.BlockSpec((1, out_dim), lambda i: (0, 0)),
        ],
        out_specs=pl.BlockSpec((blk, out_dim), lambda i: (i, 0)),
        out_shape=jax.ShapeDtypeStruct((batch, out_dim), jnp.float32),
    )


def kernel(text, embedding_table, fc_weight, fc_bias):
    seq, batch = text.shape
    vocab, dim = embedding_table.shape
    out_dim = fc_weight.shape[0]

    idxt = text.astype(jnp.int32).T
    summed = _make_gather_sum(seq, batch, vocab, dim)(idxt, embedding_table)
    fc = _make_fc(batch, dim, out_dim)
    return fc(summed, fc_weight, fc_bias.reshape(1, out_dim))
